# baseline (speedup 1.0000x reference)
import functools

import jax
import jax.numpy as jnp
from jax.experimental import pallas as pl
from jax.experimental.pallas import tpu as pltpu


def _gcn_block_kernel(a_ref, e_ref, o_ref):
    acc = jnp.dot(a_ref[...], e_ref[...], preferred_element_type=jnp.float32)
    o_ref[...] = jnp.where(acc >= 0, acc, 0.5 * acc)


@functools.partial(jax.jit, static_argnames=("block_m",))
def _gcn_matmul(adj, embeds, block_m=256):
    m, k = adj.shape
    n = embeds.shape[1]
    return pl.pallas_call(
        _gcn_block_kernel,
        grid=(m // block_m,),
        in_specs=[
            pl.BlockSpec((block_m, k), lambda i: (i, 0)),
            pl.BlockSpec((k, n), lambda i: (0, 0)),
        ],
        out_specs=pl.BlockSpec((block_m, n), lambda i: (i, 0)),
        out_shape=jax.ShapeDtypeStruct((m, n), jnp.float32),
        compiler_params=pltpu.CompilerParams(
            dimension_semantics=("parallel",),
        ),
    )(adj, embeds)


def kernel(adj, embeds):
    return _gcn_matmul(adj, embeds)
